# Initial kernel scaffold; baseline (speedup 1.0000x reference)
import jax
import jax.numpy as jnp
from jax.experimental import pallas as pl


def kernel(patch_features, coord_map):
    raise NotImplementedError("write your pallas kernel here")


# speedup vs baseline: 386.3640x; 386.3640x over previous
import functools

import jax
import jax.numpy as jnp
from jax.experimental import pallas as pl

K = 32
XMIN, XMAX = -15.0, 15.0
YMIN, YMAX = -15.0, 15.0
EPS = 1e-06


def _splat_body(cx_ref, cy_ref, f_ref, world_ref, wsum_ref, *, C, P):
    cxv = cx_ref[0]
    cyv = cy_ref[0]
    gx = (cxv - XMIN) * ((K - 1) / max(XMAX - XMIN, 1e-06))
    gy = (cyv - YMIN) * ((K - 1) / max(YMAX - YMIN, 1e-06))
    x0 = jnp.floor(gx)
    y0 = jnp.floor(gy)
    x1 = x0 + 1.0
    y1 = y0 + 1.0
    wx1 = gx - x0
    wy1 = gy - y0
    wx0 = 1.0 - wx1
    wy0 = 1.0 - wy1
    neighbors = (
        (x0, y0, wx0 * wy0),
        (x1, y0, wx1 * wy0),
        (x0, y1, wx0 * wy1),
        (x1, y1, wx1 * wy1),
    )
    ciota = jax.lax.broadcasted_iota(jnp.int32, (C, P), 0)
    M = jnp.zeros((C, P), dtype=jnp.float32)
    for nx, ny, w in neighbors:
        valid = (nx >= 0.0) & (nx < K) & (ny >= 0.0) & (ny < K)
        idx = (jnp.clip(ny, 0.0, K - 1.0) * K + jnp.clip(nx, 0.0, K - 1.0)).astype(jnp.int32)
        wv = jnp.where(valid, w, 0.0)
        M = M + jnp.where(idx == ciota, wv, 0.0)
    f = f_ref[0]
    out = jax.lax.dot_general(
        f, M, (((1,), (1,)), ((), ())), preferred_element_type=jnp.float32
    )
    ones = jnp.ones((8, P), dtype=jnp.float32)
    wsum = jax.lax.dot_general(
        ones, M, (((1,), (1,)), ((), ())), preferred_element_type=jnp.float32
    )
    recip = 1.0 / jnp.clip(wsum[0:1], EPS, None)
    world_ref[0] = out * recip
    wsum_ref[0] = wsum


def kernel(patch_features, coord_map):
    b, t, d, hp, wp = patch_features.shape
    P = hp * wp
    C = K * K
    TD = t * d
    feats = patch_features.reshape(b, TD, P)
    cx = coord_map[..., 0].reshape(b, 1, P)
    cy = coord_map[..., 1].reshape(b, 1, P)

    world, wsum = pl.pallas_call(
        functools.partial(_splat_body, C=C, P=P),
        grid=(b,),
        in_specs=[
            pl.BlockSpec((1, 1, P), lambda i: (i, 0, 0)),
            pl.BlockSpec((1, 1, P), lambda i: (i, 0, 0)),
            pl.BlockSpec((1, TD, P), lambda i: (i, 0, 0)),
        ],
        out_specs=[
            pl.BlockSpec((1, TD, C), lambda i: (i, 0, 0)),
            pl.BlockSpec((1, 8, C), lambda i: (i, 0, 0)),
        ],
        out_shape=[
            jax.ShapeDtypeStruct((b, TD, C), jnp.float32),
            jax.ShapeDtypeStruct((b, 8, C), jnp.float32),
        ],
    )(cx, cy, feats)

    world = world.reshape(b, t, d, K, K)
    weights = jnp.broadcast_to(
        wsum[:, 0, :].reshape(b, 1, 1, K, K), (b, t, 1, K, K)
    )
    return (world, weights)
